# Initial kernel scaffold; baseline (speedup 1.0000x reference)
import jax
import jax.numpy as jnp
from jax.experimental import pallas as pl


def kernel(x, edge_index, W1l, b1l, W1r, W2l, b2l, W2r):
    raise NotImplementedError("write your pallas kernel here")


# speedup vs baseline: 7.0735x; 7.0735x over previous
import functools

import jax
import jax.numpy as jnp
from jax import lax
from jax.experimental import pallas as pl
from jax.experimental.pallas import tpu as pltpu
from jax.experimental.pallas import tpu_sc as plsc

_N = 10000
_E = 160000
_DIN = 256
_H = 128
_DOUT = 256

_NW = 32
_EPW = _E // _NW
_C = 125
_NCH = _EPW // _C
_NP = 10240
_RPT = _NP // 16


def _make_segsum(D):
  mesh = plsc.VectorSubcoreMesh(core_axis_name="c", subcore_axis_name="s")

  @functools.partial(
      pl.kernel,
      out_type=jax.ShapeDtypeStruct((2, _NP, D), jnp.float32),
      mesh=mesh,
      compiler_params=pltpu.CompilerParams(use_tc_tiling_on_sc=False),
      scratch_types=[
          pltpu.VMEM((_NCH, _C), jnp.int32),
          pltpu.VMEM((_NCH, _C), jnp.int32),
          pltpu.VMEM((_C, D), jnp.float32),
          pltpu.VMEM_SHARED((_NP, D), jnp.float32),
          pltpu.SemaphoreType.DMA,
      ],
  )
  def segsum(table, srcs, dsts, zrows, out, src_v, dst_v, rows_v, acc, sem):
    c = lax.axis_index("c")
    s = lax.axis_index("s")
    wid = c * 16 + s
    pltpu.sync_copy(zrows, acc.at[pl.ds(s * _RPT, _RPT)])
    pltpu.sync_copy(srcs.at[wid], src_v)
    pltpu.sync_copy(dsts.at[wid], dst_v)
    plsc.subcore_barrier()

    def body(j, carry):
      pltpu.async_copy(table.at[src_v.at[j]], rows_v, sem).wait()
      pltpu.sync_copy(rows_v, acc.at[dst_v.at[j]], add=True)
      return carry

    lax.fori_loop(0, _NCH, body, 0)
    plsc.subcore_barrier()
    pltpu.sync_copy(acc.at[pl.ds(s * _RPT, _RPT)],
                    out.at[c, pl.ds(s * _RPT, _RPT)])

  return segsum


_segsum144 = _make_segsum(_H + 16)
_segsum128 = _make_segsum(_H)

_BR = 1000


def _pre_body(x_ref, w1lt_ref, w1rt_ref, p1ext_ref, r1_ref):
  p1 = jnp.dot(x_ref[:], w1lt_ref[:], preferred_element_type=jnp.float32)
  lane = lax.broadcasted_iota(jnp.int32, (_BR, 16), 1)
  ext = jnp.where(lane == 0, 1.0, 0.0).astype(jnp.float32)
  p1ext_ref[:] = jnp.concatenate([p1, ext], axis=1)
  r1_ref[:] = jnp.dot(x_ref[:], w1rt_ref[:], preferred_element_type=jnp.float32)


def _mid_body(p0_ref, p1_ref, r1_ref, b1l_ref, h_ref, inv_ref):
  s = p0_ref[:] + p1_ref[:]
  cnt = s[:, _H:_H + 1]
  inv = 1.0 / jnp.maximum(cnt, 1.0)
  h_ref[:] = jnp.maximum(s[:, :_H] * inv + b1l_ref[:] + r1_ref[:], 0.0)
  inv_ref[:] = jnp.broadcast_to(inv, (_BR, _H))


def _post_body(p0_ref, p1_ref, inv_ref, h_ref, w2lt_ref, b2l_ref, w2rt_ref,
               z_ref):
  mean2 = (p0_ref[:] + p1_ref[:]) * inv_ref[:]
  z_ref[:] = (jnp.dot(mean2, w2lt_ref[:], preferred_element_type=jnp.float32)
              + b2l_ref[:]
              + jnp.dot(h_ref[:], w2rt_ref[:],
                        preferred_element_type=jnp.float32))


def _row_spec(d):
  return pl.BlockSpec((_BR, d), lambda i: (i, 0))


def _full_spec(r, d):
  return pl.BlockSpec((r, d), lambda i: (0, 0))


_pre = pl.pallas_call(
    _pre_body,
    grid=(_N // _BR,),
    in_specs=[_row_spec(_DIN), _full_spec(_DIN, _H), _full_spec(_DIN, _H)],
    out_specs=[_row_spec(_H + 16), _row_spec(_H)],
    out_shape=[
        jax.ShapeDtypeStruct((_N, _H + 16), jnp.float32),
        jax.ShapeDtypeStruct((_N, _H), jnp.float32),
    ],
)

_mid = pl.pallas_call(
    _mid_body,
    grid=(_N // _BR,),
    in_specs=[_row_spec(_H + 16), _row_spec(_H + 16), _row_spec(_H),
              _full_spec(1, _H)],
    out_specs=[_row_spec(_H), _row_spec(_H)],
    out_shape=[
        jax.ShapeDtypeStruct((_N, _H), jnp.float32),
        jax.ShapeDtypeStruct((_N, _H), jnp.float32),
    ],
)

_post = pl.pallas_call(
    _post_body,
    grid=(_N // _BR,),
    in_specs=[_row_spec(_H), _row_spec(_H), _row_spec(_H), _row_spec(_H),
              _full_spec(_H, _DOUT), _full_spec(1, _DOUT),
              _full_spec(_H, _DOUT)],
    out_specs=_row_spec(_DOUT),
    out_shape=jax.ShapeDtypeStruct((_N, _DOUT), jnp.float32),
)


@jax.jit
def kernel(x, edge_index, W1l, b1l, W1r, W2l, b2l, W2r):
  src = edge_index[0].reshape(_NW, _NCH, _C)
  dst = edge_index[1].reshape(_NW, _NCH, _C)
  z144 = jnp.zeros((_RPT, _H + 16), jnp.float32)
  z128 = jnp.zeros((_RPT, _H), jnp.float32)

  p1ext, r1 = _pre(x, W1l.T, W1r.T)
  part1 = _segsum144(p1ext, src, dst, z144)
  h, inv = _mid(part1[0], part1[1], r1, b1l.reshape(1, _H))
  part2 = _segsum128(h, src, dst, z128)
  z = _post(part2[0], part2[1], inv, h, W2l.T, b2l.reshape(1, _DOUT), W2r.T)
  return z


# speedup vs baseline: 8.8529x; 1.2516x over previous
import functools

import jax
import jax.numpy as jnp
from jax import lax
from jax.experimental import pallas as pl
from jax.experimental.pallas import tpu as pltpu
from jax.experimental.pallas import tpu_sc as plsc

_N = 10000
_E = 160000
_DIN = 256
_H = 128
_DOUT = 256

_NW = 32
_EPW = _E // _NW
_C = 100
_NCH = _EPW // _C
_NP = _N
_RPT = _NP // 16


def _make_segsum(D):
  mesh = plsc.VectorSubcoreMesh(core_axis_name="c", subcore_axis_name="s")

  @functools.partial(
      pl.kernel,
      out_type=jax.ShapeDtypeStruct((2, _NP, D), jnp.float32),
      mesh=mesh,
      compiler_params=pltpu.CompilerParams(use_tc_tiling_on_sc=False),
      scratch_types=[
          pltpu.VMEM((_NCH, _C), jnp.int32),
          pltpu.VMEM((_NCH, _C), jnp.int32),
          pltpu.VMEM((_C, D), jnp.float32),
          pltpu.VMEM((_C, D), jnp.float32),
          pltpu.VMEM_SHARED((_NP, D), jnp.float32),
          pltpu.SemaphoreType.DMA,
          pltpu.SemaphoreType.DMA,
      ],
  )
  def segsum(table, srcs, dsts, zrows, out, src_v, dst_v, rows0_v, rows1_v,
             acc, sem0, sem1):
    c = lax.axis_index("c")
    s = lax.axis_index("s")
    wid = c * 16 + s
    pltpu.sync_copy(zrows, acc.at[pl.ds(s * _RPT, _RPT)])
    pltpu.sync_copy(srcs.at[wid], src_v)
    pltpu.sync_copy(dsts.at[wid], dst_v)
    plsc.subcore_barrier()

    pltpu.async_copy(table.at[src_v.at[0]], rows0_v, sem0)

    def body(k, carry):
      j0 = 2 * k
      pltpu.async_copy(table.at[src_v.at[j0 + 1]], rows1_v, sem1)
      pltpu.make_async_copy(table.at[src_v.at[j0]], rows0_v, sem0).wait()
      pltpu.sync_copy(rows0_v, acc.at[dst_v.at[j0]], add=True)

      @pl.when(k < _NCH // 2 - 1)
      def _():
        pltpu.async_copy(table.at[src_v.at[j0 + 2]], rows0_v, sem0)

      pltpu.make_async_copy(table.at[src_v.at[j0 + 1]], rows1_v, sem1).wait()
      pltpu.sync_copy(rows1_v, acc.at[dst_v.at[j0 + 1]], add=True)
      return carry

    lax.fori_loop(0, _NCH // 2, body, 0)
    plsc.subcore_barrier()
    pltpu.sync_copy(acc.at[pl.ds(s * _RPT, _RPT)],
                    out.at[c, pl.ds(s * _RPT, _RPT)])

  return segsum


_segsum144 = _make_segsum(_H + 16)
_segsum128 = _make_segsum(_H)

_BR = 1000


def _pre_body(x_ref, w1lt_ref, w1rt_ref, p1ext_ref, r1_ref):
  p1 = jnp.dot(x_ref[:], w1lt_ref[:], preferred_element_type=jnp.float32)
  lane = lax.broadcasted_iota(jnp.int32, (_BR, 16), 1)
  ext = jnp.where(lane == 0, 1.0, 0.0).astype(jnp.float32)
  p1ext_ref[:] = jnp.concatenate([p1, ext], axis=1)
  r1_ref[:] = jnp.dot(x_ref[:], w1rt_ref[:], preferred_element_type=jnp.float32)


def _mid_body(p0_ref, p1_ref, r1_ref, b1l_ref, h_ref, inv_ref):
  s = p0_ref[:] + p1_ref[:]
  cnt = s[:, _H:_H + 1]
  inv = 1.0 / jnp.maximum(cnt, 1.0)
  h_ref[:] = jnp.maximum(s[:, :_H] * inv + b1l_ref[:] + r1_ref[:], 0.0)
  inv_ref[:] = jnp.broadcast_to(inv, (_BR, _H))


def _post_body(p0_ref, p1_ref, inv_ref, h_ref, w2lt_ref, b2l_ref, w2rt_ref,
               z_ref):
  mean2 = (p0_ref[:] + p1_ref[:]) * inv_ref[:]
  z_ref[:] = (jnp.dot(mean2, w2lt_ref[:], preferred_element_type=jnp.float32)
              + b2l_ref[:]
              + jnp.dot(h_ref[:], w2rt_ref[:],
                        preferred_element_type=jnp.float32))


def _row_spec(d):
  return pl.BlockSpec((_BR, d), lambda i: (i, 0))


def _full_spec(r, d):
  return pl.BlockSpec((r, d), lambda i: (0, 0))


_pre = pl.pallas_call(
    _pre_body,
    grid=(_N // _BR,),
    in_specs=[_row_spec(_DIN), _full_spec(_DIN, _H), _full_spec(_DIN, _H)],
    out_specs=[_row_spec(_H + 16), _row_spec(_H)],
    out_shape=[
        jax.ShapeDtypeStruct((_N, _H + 16), jnp.float32),
        jax.ShapeDtypeStruct((_N, _H), jnp.float32),
    ],
)

_mid = pl.pallas_call(
    _mid_body,
    grid=(_N // _BR,),
    in_specs=[_row_spec(_H + 16), _row_spec(_H + 16), _row_spec(_H),
              _full_spec(1, _H)],
    out_specs=[_row_spec(_H), _row_spec(_H)],
    out_shape=[
        jax.ShapeDtypeStruct((_N, _H), jnp.float32),
        jax.ShapeDtypeStruct((_N, _H), jnp.float32),
    ],
)

_post = pl.pallas_call(
    _post_body,
    grid=(_N // _BR,),
    in_specs=[_row_spec(_H), _row_spec(_H), _row_spec(_H), _row_spec(_H),
              _full_spec(_H, _DOUT), _full_spec(1, _DOUT),
              _full_spec(_H, _DOUT)],
    out_specs=_row_spec(_DOUT),
    out_shape=jax.ShapeDtypeStruct((_N, _DOUT), jnp.float32),
)


@jax.jit
def kernel(x, edge_index, W1l, b1l, W1r, W2l, b2l, W2r):
  src = edge_index[0].reshape(_NW, _NCH, _C)
  dst = edge_index[1].reshape(_NW, _NCH, _C)
  z144 = jnp.zeros((_RPT, _H + 16), jnp.float32)
  z128 = jnp.zeros((_RPT, _H), jnp.float32)

  p1ext, r1 = _pre(x, W1l.T, W1r.T)
  part1 = _segsum144(p1ext, src, dst, z144)
  h, inv = _mid(part1[0], part1[1], r1, b1l.reshape(1, _H))
  part2 = _segsum128(h, src, dst, z128)
  z = _post(part2[0], part2[1], inv, h, W2l.T, b2l.reshape(1, _DOUT), W2r.T)
  return z
